# baseline (speedup 1.0000x reference)
import jax
import jax.numpy as jnp
from jax.experimental import pallas as pl
from jax.experimental.pallas import tpu as pltpu

D_MODEL = 1024
D_FF = 4096
NUM_OPS = 4
T = 2 * 2048
NW = 8
FW = D_FF // NW
XW = T // NW
NT = 8
TB = T // NT


def _argmax4(op_ref):
    best = op_ref[0]
    arg = jnp.int32(0)
    for k in range(1, NUM_OPS):
        v = op_ref[k]
        take = v > best
        arg = jnp.where(take, jnp.int32(k), arg)
        best = jnp.where(take, v, best)
    return arg


def _expert(op_ref, s):
    return 2 * _argmax4(op_ref) + s


def _ffn_kernel(op_ref, x_ref, w1_ref, w2_ref, b1_ref, b2_ref, out_ref,
                w1bf_ref, w2bf_ref, xcur_ref, hbf_ref):
    s = pl.program_id(0)
    t = pl.program_id(1)

    @pl.when(t < NW)
    def _():
        w1bf_ref[:, pl.ds(t * FW, FW)] = w1_ref[0].astype(jnp.bfloat16)
        w2bf_ref[pl.ds(t * FW, FW), :] = w2_ref[0].astype(jnp.bfloat16)

        @pl.when(s == 0)
        def _():
            xcur_ref[pl.ds(t * XW, XW), :] = x_ref[...].astype(jnp.bfloat16)

    @pl.when(t > NW)
    def _():
        tbp = t - NW - 1
        hprev = hbf_ref[pl.ds((tbp % 2) * TB, TB), :]
        y = jnp.dot(hprev, w2bf_ref[...],
                    preferred_element_type=jnp.float32) + b2_ref[0, 0]

        @pl.when(s == 0)
        def _():
            xcur_ref[pl.ds(tbp * TB, TB), :] = y.astype(jnp.bfloat16)

        @pl.when(s != 0)
        def _():
            out_ref[...] = y

    @pl.when(jnp.logical_and(t >= NW, t < NW + NT))
    def _():
        tb = t - NW
        xin = xcur_ref[pl.ds(tb * TB, TB), :]
        h = jnp.dot(xin, w1bf_ref[...],
                    preferred_element_type=jnp.float32)
        h = jnp.maximum(h + b1_ref[0, 0], 0.0).astype(jnp.bfloat16)
        hbf_ref[pl.ds((tb % 2) * TB, TB), :] = h


def kernel(x, W1, b1, W2, b2):
    x2d = x.reshape(T, D_MODEL)
    opcode_scores = jax.lax.slice(x2d, (0, 0), (1, NUM_OPS)).reshape(NUM_OPS)
    b1r = b1.reshape(b1.shape[0], 1, D_FF)
    b2r = b2.reshape(b2.shape[0], 1, D_MODEL)

    out = pl.pallas_call(
        _ffn_kernel,
        grid_spec=pltpu.PrefetchScalarGridSpec(
            num_scalar_prefetch=1,
            grid=(2, NW + NT + 1),
            in_specs=[
                pl.BlockSpec((XW, D_MODEL),
                             lambda s, t, op: (jnp.minimum(t, NW - 1), 0)),
                pl.BlockSpec((1, D_MODEL, FW),
                             lambda s, t, op: (_expert(op, s), 0,
                                               jnp.minimum(t, NW - 1))),
                pl.BlockSpec((1, FW, D_MODEL),
                             lambda s, t, op: (_expert(op, s),
                                               jnp.minimum(t, NW - 1), 0)),
                pl.BlockSpec((1, 1, D_FF),
                             lambda s, t, op: (_expert(op, s), 0, 0)),
                pl.BlockSpec((1, 1, D_MODEL),
                             lambda s, t, op: (_expert(op, s), 0, 0)),
            ],
            out_specs=pl.BlockSpec(
                (TB, D_MODEL),
                lambda s, t, op: (jnp.where(s == 0, 0,
                                            jnp.clip(t - NW - 1, 0, NT - 1)),
                                  0)),
            scratch_shapes=[
                pltpu.VMEM((D_MODEL, D_FF), jnp.bfloat16),
                pltpu.VMEM((D_FF, D_MODEL), jnp.bfloat16),
                pltpu.VMEM((T, D_MODEL), jnp.bfloat16),
                pltpu.VMEM((2 * TB, D_FF), jnp.bfloat16),
            ],
        ),
        out_shape=jax.ShapeDtypeStruct((T, D_MODEL), jnp.float32),
        compiler_params=pltpu.CompilerParams(
            dimension_semantics=("arbitrary", "arbitrary")),
    )(opcode_scores, x2d, W1, W2, b1r, b2r)
    return out.reshape(x.shape)


# speedup vs baseline: 1.0237x; 1.0237x over previous
import jax
import jax.numpy as jnp
from jax.experimental import pallas as pl
from jax.experimental.pallas import tpu as pltpu

D_MODEL = 1024
D_FF = 4096
NUM_OPS = 4
T = 2 * 2048
NW = 8
FW = D_FF // NW
XW = T // NW
NT = 8
TB = T // NT
NC = 4
FC = D_FF // NC


def _argmax4(op_ref):
    best = op_ref[0]
    arg = jnp.int32(0)
    for k in range(1, NUM_OPS):
        v = op_ref[k]
        take = v > best
        arg = jnp.where(take, jnp.int32(k), arg)
        best = jnp.where(take, v, best)
    return arg


def _expert(op_ref, s):
    return 2 * _argmax4(op_ref) + s


def _ffn_kernel(op_ref, x_ref, w1_ref, w2_ref, b1_ref, b2_ref, out_ref,
                w1bf_ref, w2bf_ref, xcur_ref):
    s = pl.program_id(0)
    t = pl.program_id(1)

    @pl.when(t < NW)
    def _():
        w1bf_ref[:, pl.ds(t * FW, FW)] = w1_ref[0].astype(jnp.bfloat16)
        w2bf_ref[pl.ds(t * FW, FW), :] = w2_ref[0].astype(jnp.bfloat16)

        @pl.when(s == 0)
        def _():
            xcur_ref[pl.ds(t * XW, XW), :] = x_ref[...].astype(jnp.bfloat16)

    @pl.when(t >= NW)
    def _():
        tb = t - NW
        xin = xcur_ref[pl.ds(tb * TB, TB), :]
        b1v = b1_ref[0, 0]
        ys = []
        for c in range(NC):
            lo = c * FC
            hc = jnp.dot(xin, w1bf_ref[:, lo:lo + FC],
                         preferred_element_type=jnp.float32)
            hc = jnp.maximum(hc + b1v[lo:lo + FC], 0.0).astype(jnp.bfloat16)
            ys.append(jnp.dot(hc, w2bf_ref[lo:lo + FC, :],
                              preferred_element_type=jnp.float32))
        y = (ys[0] + ys[1]) + (ys[2] + ys[3]) + b2_ref[0, 0]

        @pl.when(s == 0)
        def _():
            xcur_ref[pl.ds(tb * TB, TB), :] = y.astype(jnp.bfloat16)

        @pl.when(s != 0)
        def _():
            out_ref[...] = y


def kernel(x, W1, b1, W2, b2):
    x2d = x.reshape(T, D_MODEL)
    opcode_scores = jax.lax.slice(x2d, (0, 0), (1, NUM_OPS)).reshape(NUM_OPS)
    b1r = b1.reshape(b1.shape[0], 1, D_FF)
    b2r = b2.reshape(b2.shape[0], 1, D_MODEL)

    out = pl.pallas_call(
        _ffn_kernel,
        grid_spec=pltpu.PrefetchScalarGridSpec(
            num_scalar_prefetch=1,
            grid=(2, NW + NT),
            in_specs=[
                pl.BlockSpec((XW, D_MODEL),
                             lambda s, t, op: (jnp.minimum(t, NW - 1), 0)),
                pl.BlockSpec((1, D_MODEL, FW),
                             lambda s, t, op: (_expert(op, s), 0,
                                               jnp.minimum(t, NW - 1))),
                pl.BlockSpec((1, FW, D_MODEL),
                             lambda s, t, op: (_expert(op, s),
                                               jnp.minimum(t, NW - 1), 0)),
                pl.BlockSpec((1, 1, D_FF),
                             lambda s, t, op: (_expert(op, s), 0, 0)),
                pl.BlockSpec((1, 1, D_MODEL),
                             lambda s, t, op: (_expert(op, s), 0, 0)),
            ],
            out_specs=pl.BlockSpec(
                (TB, D_MODEL),
                lambda s, t, op: (jnp.where(s == 0, 0,
                                            jnp.clip(t - NW, 0, NT - 1)),
                                  0)),
            scratch_shapes=[
                pltpu.VMEM((D_MODEL, D_FF), jnp.bfloat16),
                pltpu.VMEM((D_FF, D_MODEL), jnp.bfloat16),
                pltpu.VMEM((T, D_MODEL), jnp.bfloat16),
            ],
        ),
        out_shape=jax.ShapeDtypeStruct((T, D_MODEL), jnp.float32),
        compiler_params=pltpu.CompilerParams(
            dimension_semantics=("arbitrary", "arbitrary")),
    )(opcode_scores, x2d, W1, W2, b1r, b2r)
    return out.reshape(x.shape)
